# baseline (speedup 1.0000x reference)
import jax
import jax.numpy as jnp
from jax import lax
from jax.experimental import pallas as pl
from jax.experimental.pallas import tpu as pltpu
from jax.experimental.pallas import tpu_sc as plsc

B = 16384
V = 5
C = 5
NC = 2
NS = 16
L = 16
NW = NC * NS
ROWS_PER_TILE = B // NW
VECS_PER_TILE = ROWS_PER_TILE // L
OUT_PER_TILE = ROWS_PER_TILE * C

_LN2 = 0.6931471805599453
_MAGIC = 0x3F3504F3

_DNUMS = lax.GatherDimensionNumbers(
    offset_dims=(), collapsed_slice_dims=(0,), start_index_map=(0,))


def _dyng(tab, idx):
    return lax.gather(tab, idx[:, None], _DNUMS, (1,),
                      mode=lax.GatherScatterMode.PROMISE_IN_BOUNDS)


def _vlog(x):
    xi = plsc.bitcast(x, jnp.int32)
    e = lax.shift_right_arithmetic(xi - _MAGIC, 23)
    m = plsc.bitcast(xi - lax.shift_left(e, 23), jnp.float32)
    z = (m - 1.0) / (m + 1.0)
    z2 = z * z
    return e.astype(jnp.float32) * _LN2 + z * (2.0 + z2 * (0.66666667 + z2 * 0.4))


def _body(w_hbm, vt_hbm, af_hbm, out_hbm, vt_v, af_v, w_v, out_v,
          sem_io, sem_w):
    wid = lax.axis_index("s") * NC + lax.axis_index("c")
    base = wid * ROWS_PER_TILE
    cp1 = pltpu.async_copy(vt_hbm.at[pl.ds(base, ROWS_PER_TILE)], vt_v, sem_io)
    cp2 = pltpu.async_copy(af_hbm.at[pl.ds(base, ROWS_PER_TILE)], af_v, sem_io)
    cp3 = pltpu.async_copy(w_hbm, w_v, sem_w)
    cp3.wait()

    lane = lax.iota(jnp.int32, L)
    vidx = jnp.minimum(lane, V - 1) * C
    at = plsc.load_gather(w_v, [vidx])
    rt = plsc.load_gather(w_v, [vidx + 1])
    r2t = plsc.load_gather(w_v, [vidx + 4])
    mt = jnp.maximum(jnp.maximum(at, rt), jnp.maximum(r2t, 0.0))
    emt = jnp.exp(0.0 - mt)
    et = jnp.exp(at - mt) + jnp.exp(rt - mt) + emt + jnp.exp(r2t - mt)

    cp1.wait()
    cp2.wait()

    lane5 = lane * C

    def step(i, carry):
        v16 = vt_v[pl.ds(i * L, L)]
        af = af_v[pl.ds(i * L, L)]
        a = _dyng(at, v16)
        r = _dyng(rt, v16)
        r2 = _dyng(r2t, v16)
        m = _dyng(mt, v16)
        em = _dyng(emt, v16)
        e = _dyng(et, v16)
        p = af * (2.0 - af)
        g = _vlog(p)
        s = e + p * em
        nrm = m + _vlog(s)
        off = lane5 + i * (L * C)
        plsc.store_scatter(out_v, [off], a - nrm)
        plsc.store_scatter(out_v, [off + 1], r - nrm)
        plsc.store_scatter(out_v, [off + 2], 0.0 - nrm)
        plsc.store_scatter(out_v, [off + 3], g - nrm)
        plsc.store_scatter(out_v, [off + 4], r2 - nrm)
        return carry

    lax.fori_loop(0, VECS_PER_TILE, step, 0, unroll=8)
    pltpu.sync_copy(out_v, out_hbm.at[pl.ds(base * C, OUT_PER_TILE)])


@jax.jit
def _posterior_priors_sc(w, vt, af):
    mesh = plsc.VectorSubcoreMesh(core_axis_name="c", subcore_axis_name="s",
                                  num_cores=NC, num_subcores=NS)
    flat = pl.kernel(
        _body,
        out_type=jax.ShapeDtypeStruct((B * C,), jnp.float32),
        mesh=mesh,
        scratch_types=[
            pltpu.VMEM((ROWS_PER_TILE,), jnp.int32),
            pltpu.VMEM((ROWS_PER_TILE,), jnp.float32),
            pltpu.VMEM((V * C,), jnp.float32),
            pltpu.VMEM((OUT_PER_TILE,), jnp.float32),
            pltpu.SemaphoreType.DMA,
            pltpu.SemaphoreType.DMA,
        ],
        compiler_params=pltpu.CompilerParams(
            needs_layout_passes=False,
            disable_bounds_checks=True,
            disable_semaphore_checks=True,
            skip_device_barrier=True,
        ),
    )(w.reshape(V * C), vt, af)
    return flat.reshape(B, C)


def kernel(variant_types_b, allele_frequencies_1d, unnormalized_priors_vc):
    return _posterior_priors_sc(unnormalized_priors_vc, variant_types_b,
                                allele_frequencies_1d)


# speedup vs baseline: 1.2254x; 1.2254x over previous
import jax
import jax.numpy as jnp
from jax import lax
from jax.experimental import pallas as pl
from jax.experimental.pallas import tpu as pltpu
from jax.experimental.pallas import tpu_sc as plsc

B = 16384
V = 5
C = 5
NC = 2
NS = 16
L = 16
NW = NC * NS
ROWS_PER_TILE = B // NW
VECS_PER_TILE = ROWS_PER_TILE // L
OUT_PER_TILE = ROWS_PER_TILE * C

_LN2 = 0.6931471805599453
_MAGIC = 0x3F3504F3

_DNUMS = lax.GatherDimensionNumbers(
    offset_dims=(), collapsed_slice_dims=(0,), start_index_map=(0,))


def _dyng(tab, idx):
    return lax.gather(tab, idx[:, None], _DNUMS, (1,),
                      mode=lax.GatherScatterMode.PROMISE_IN_BOUNDS)


def _vlog(x):
    xi = plsc.bitcast(x, jnp.int32)
    e = lax.shift_right_arithmetic(xi - _MAGIC, 23)
    m = plsc.bitcast(xi - lax.shift_left(e, 23), jnp.float32)
    z = (m - 1.0) / (m + 1.0)
    z2 = z * z
    return e.astype(jnp.float32) * _LN2 + z * (2.0 + z2 * (0.66666667 + z2 * 0.4))


def _body(w_hbm, vt_hbm, af_hbm, out_hbm, vt_v, af_v, w_v, out_v,
          sem_io, sem_w):
    wid = lax.axis_index("s") * NC + lax.axis_index("c")
    base = wid * ROWS_PER_TILE
    cp1 = pltpu.async_copy(vt_hbm.at[pl.ds(base, ROWS_PER_TILE)], vt_v, sem_io)
    cp2 = pltpu.async_copy(af_hbm.at[pl.ds(base, ROWS_PER_TILE)], af_v, sem_io)
    cp3 = pltpu.async_copy(w_hbm, w_v, sem_w)
    cp3.wait()

    lane = lax.iota(jnp.int32, L)
    vidx = jnp.minimum(lane, V - 1) * C
    at = plsc.load_gather(w_v, [vidx])
    rt = plsc.load_gather(w_v, [vidx + 1])
    r2t = plsc.load_gather(w_v, [vidx + 4])
    mt = jnp.maximum(jnp.maximum(at, rt), jnp.maximum(r2t, 0.0))
    emt = jnp.exp(0.0 - mt)
    et = jnp.exp(at - mt) + jnp.exp(rt - mt) + emt + jnp.exp(r2t - mt)

    cp1.wait()
    cp2.wait()

    c0 = jnp.zeros((L,), jnp.int32)
    c1 = c0 + 1
    c2 = c0 + 2
    c3 = c0 + 3
    c4 = c0 + 4

    def step(i, carry):
        v16 = vt_v[pl.ds(i * L, L)]
        af = af_v[pl.ds(i * L, L)]
        a = _dyng(at, v16)
        r = _dyng(rt, v16)
        r2 = _dyng(r2t, v16)
        m = _dyng(mt, v16)
        em = _dyng(emt, v16)
        e = _dyng(et, v16)
        p = af * (2.0 - af)
        g = _vlog(p)
        s = e + p * em
        nrm = m + _vlog(s)
        row = lane + i * L
        plsc.store_scatter(out_v, [row, c0], a - nrm)
        plsc.store_scatter(out_v, [row, c1], r - nrm)
        plsc.store_scatter(out_v, [row, c2], 0.0 - nrm)
        plsc.store_scatter(out_v, [row, c3], g - nrm)
        plsc.store_scatter(out_v, [row, c4], r2 - nrm)
        return carry

    lax.fori_loop(0, VECS_PER_TILE, step, 0, unroll=8)
    pltpu.sync_copy(out_v, out_hbm.at[pl.ds(base, ROWS_PER_TILE), :])


@jax.jit
def _posterior_priors_sc(w, vt, af):
    mesh = plsc.VectorSubcoreMesh(core_axis_name="c", subcore_axis_name="s",
                                  num_cores=NC, num_subcores=NS)
    return pl.kernel(
        _body,
        out_type=jax.ShapeDtypeStruct((B, C), jnp.float32),
        mesh=mesh,
        scratch_types=[
            pltpu.VMEM((ROWS_PER_TILE,), jnp.int32),
            pltpu.VMEM((ROWS_PER_TILE,), jnp.float32),
            pltpu.VMEM((V * C,), jnp.float32),
            pltpu.VMEM((ROWS_PER_TILE, C), jnp.float32),
            pltpu.SemaphoreType.DMA,
            pltpu.SemaphoreType.DMA,
        ],
        compiler_params=pltpu.CompilerParams(
            needs_layout_passes=False,
            disable_bounds_checks=True,
            disable_semaphore_checks=True,
            skip_device_barrier=True,
        ),
    )(w.reshape(V * C), vt, af)


def kernel(variant_types_b, allele_frequencies_1d, unnormalized_priors_vc):
    return _posterior_priors_sc(unnormalized_priors_vc, variant_types_b,
                                allele_frequencies_1d)


# speedup vs baseline: 1.2325x; 1.0058x over previous
import jax
import jax.numpy as jnp
from jax import lax
from jax.experimental import pallas as pl
from jax.experimental.pallas import tpu as pltpu
from jax.experimental.pallas import tpu_sc as plsc

B = 16384
V = 5
C = 5
NC = 2
NS = 16
L = 16
NW = NC * NS
ROWS_PER_TILE = B // NW
VECS_PER_TILE = ROWS_PER_TILE // L
OUT_PER_TILE = ROWS_PER_TILE * C

_LN2 = 0.6931471805599453
_MAGIC = 0x3F3504F3

_DNUMS = lax.GatherDimensionNumbers(
    offset_dims=(), collapsed_slice_dims=(0,), start_index_map=(0,))


def _dyng(tab, idx):
    return lax.gather(tab, idx[:, None], _DNUMS, (1,),
                      mode=lax.GatherScatterMode.PROMISE_IN_BOUNDS)


def _vlog(x):
    xi = plsc.bitcast(x, jnp.int32)
    e = lax.shift_right_arithmetic(xi - _MAGIC, 23)
    m = plsc.bitcast(xi - lax.shift_left(e, 23), jnp.float32)
    z = (m - 1.0) / (m + 1.0)
    z2 = z * z
    return e.astype(jnp.float32) * _LN2 + z * (2.0 + z2 * (0.66666667 + z2 * 0.4))


def _body(w_hbm, vt_hbm, af_hbm, out_hbm, vt_v, af_v, w_v, out_v,
          sem_io, sem_w):
    wid = lax.axis_index("s") * NC + lax.axis_index("c")
    base = wid * ROWS_PER_TILE
    cp1 = pltpu.async_copy(vt_hbm.at[pl.ds(base, ROWS_PER_TILE)], vt_v, sem_io)
    cp2 = pltpu.async_copy(af_hbm.at[pl.ds(base, ROWS_PER_TILE)], af_v, sem_io)
    cp3 = pltpu.async_copy(w_hbm, w_v, sem_w)
    cp3.wait()

    lane = lax.iota(jnp.int32, L)
    vidx = jnp.minimum(lane, V - 1) * C
    at = plsc.load_gather(w_v, [vidx])
    rt = plsc.load_gather(w_v, [vidx + 1])
    r2t = plsc.load_gather(w_v, [vidx + 4])
    mt = jnp.maximum(jnp.maximum(at, rt), jnp.maximum(r2t, 0.0))
    emt = jnp.exp(0.0 - mt)
    et = jnp.exp(at - mt) + jnp.exp(rt - mt) + emt + jnp.exp(r2t - mt)

    cp1.wait()
    cp2.wait()

    c0 = jnp.zeros((L,), jnp.int32)
    c1 = c0 + 1
    c2 = c0 + 2
    c3 = c0 + 3
    c4 = c0 + 4

    def step(i, carry):
        v16 = vt_v[pl.ds(i * L, L)]
        af = af_v[pl.ds(i * L, L)]
        a = _dyng(at, v16)
        r = _dyng(rt, v16)
        r2 = _dyng(r2t, v16)
        m = _dyng(mt, v16)
        em = _dyng(emt, v16)
        e = _dyng(et, v16)
        p = af * (2.0 - af)
        g = _vlog(p)
        s = e + p * em
        nrm = m + _vlog(s)
        row = lane + i * L
        plsc.store_scatter(out_v, [row, c0], a - nrm)
        plsc.store_scatter(out_v, [row, c1], r - nrm)
        plsc.store_scatter(out_v, [row, c2], 0.0 - nrm)
        plsc.store_scatter(out_v, [row, c3], g - nrm)
        plsc.store_scatter(out_v, [row, c4], r2 - nrm)
        return carry

    H = ROWS_PER_TILE // 2
    lax.fori_loop(0, VECS_PER_TILE // 2, step, 0, unroll=8)
    cpo1 = pltpu.async_copy(out_v.at[pl.ds(0, H), :],
                            out_hbm.at[pl.ds(base, H), :], sem_w)
    lax.fori_loop(VECS_PER_TILE // 2, VECS_PER_TILE, step, 0, unroll=8)
    cpo2 = pltpu.async_copy(out_v.at[pl.ds(H, H), :],
                            out_hbm.at[pl.ds(base + H, H), :], sem_io)
    cpo1.wait()
    cpo2.wait()


@jax.jit
def _posterior_priors_sc(w, vt, af):
    mesh = plsc.VectorSubcoreMesh(core_axis_name="c", subcore_axis_name="s",
                                  num_cores=NC, num_subcores=NS)
    return pl.kernel(
        _body,
        out_type=jax.ShapeDtypeStruct((B, C), jnp.float32),
        mesh=mesh,
        scratch_types=[
            pltpu.VMEM((ROWS_PER_TILE,), jnp.int32),
            pltpu.VMEM((ROWS_PER_TILE,), jnp.float32),
            pltpu.VMEM((V * C,), jnp.float32),
            pltpu.VMEM((ROWS_PER_TILE, C), jnp.float32),
            pltpu.SemaphoreType.DMA,
            pltpu.SemaphoreType.DMA,
        ],
        compiler_params=pltpu.CompilerParams(
            needs_layout_passes=False,
            disable_bounds_checks=True,
            disable_semaphore_checks=True,
            skip_device_barrier=True,
        ),
    )(w.reshape(V * C), vt, af)


def kernel(variant_types_b, allele_frequencies_1d, unnormalized_priors_vc):
    return _posterior_priors_sc(unnormalized_priors_vc, variant_types_b,
                                allele_frequencies_1d)
